# initial kernel scaffold (unmeasured)
import jax
import jax.numpy as jnp
from jax import lax
from jax.experimental import pallas as pl
from jax.experimental.pallas import tpu as pltpu

N_DEV = 4


def kernel(x, W1, W2):
    m, k_in = x.shape
    _, h_per = W1.shape
    _, n = W2.shape

    def body(x_ref, w1_ref, w2_ref, out_ref, comm_ref, send_sems, recv_sems):
        my = lax.axis_index("i")
        left = lax.rem(my + N_DEV - 1, N_DEV)
        right = lax.rem(my + 1, N_DEV)

        barrier_sem = pltpu.get_barrier_semaphore()
        for nbr in (left, right):
            pl.semaphore_signal(
                barrier_sem, inc=1,
                device_id=(nbr,), device_id_type=pl.DeviceIdType.MESH,
            )
        pl.semaphore_wait(barrier_sem, 2)

        h = jnp.maximum(
            jnp.dot(x_ref[...], w1_ref[...], preferred_element_type=jnp.float32),
            0.0,
        )
        partial = jnp.dot(h, w2_ref[...], preferred_element_type=jnp.float32)
        out_ref[...] = partial
        comm_ref[0] = partial

        for hop in range(N_DEV - 1):
            s = hop % 2
            r = (hop + 1) % 2
            rdma = pltpu.make_async_remote_copy(
                src_ref=comm_ref.at[s],
                dst_ref=comm_ref.at[r],
                send_sem=send_sems.at[s],
                recv_sem=recv_sems.at[r],
                device_id=(right,),
                device_id_type=pl.DeviceIdType.MESH,
            )
            rdma.start()
            rdma.wait()
            out_ref[...] += comm_ref[r]

    return pl.pallas_call(
        body,
        out_shape=jax.ShapeDtypeStruct((m, n), jnp.float32),
        in_specs=[
            pl.BlockSpec(memory_space=pltpu.VMEM),
            pl.BlockSpec(memory_space=pltpu.VMEM),
            pl.BlockSpec(memory_space=pltpu.VMEM),
        ],
        out_specs=pl.BlockSpec(memory_space=pltpu.VMEM),
        scratch_shapes=[
            pltpu.VMEM((2, m, n), jnp.float32),
            pltpu.SemaphoreType.DMA((2,)),
            pltpu.SemaphoreType.DMA((2,)),
        ],
        compiler_params=pltpu.CompilerParams(collective_id=0),
    )(x, W1, W2)


# baseline (device time: 382466 ns/iter reference)
import jax
import jax.numpy as jnp
from jax import lax
from jax.experimental import pallas as pl
from jax.experimental.pallas import tpu as pltpu

N_DEV = 4
KC = 256


def kernel(x, W1, W2):
    m, k_in = x.shape
    _, h_per = W1.shape
    _, n = W2.shape
    nk = h_per // KC

    def body(x_ref, w1_hbm, w2_hbm, out_ref,
             w1_buf, w2_buf, load_sems, comm_ref, send_sems, recv_sems):
        my = lax.axis_index("i")
        left = lax.rem(my + N_DEV - 1, N_DEV)
        right = lax.rem(my + 1, N_DEV)

        barrier_sem = pltpu.get_barrier_semaphore()
        for nbr in (left, right):
            pl.semaphore_signal(
                barrier_sem, inc=1,
                device_id=(nbr,), device_id_type=pl.DeviceIdType.MESH,
            )
        pl.semaphore_wait(barrier_sem, 2)

        def load(kc, slot):
            c1 = pltpu.make_async_copy(
                w1_hbm.at[:, pl.ds(kc * KC, KC)], w1_buf.at[slot],
                load_sems.at[slot, 0])
            c2 = pltpu.make_async_copy(
                w2_hbm.at[pl.ds(kc * KC, KC), :], w2_buf.at[slot],
                load_sems.at[slot, 1])
            c1.start()
            c2.start()
            return c1, c2

        pending = load(0, 0)
        for kc in range(nk):
            slot = kc % 2
            if kc + 1 < nk:
                nxt = load(kc + 1, (kc + 1) % 2)
            pending[0].wait()
            pending[1].wait()
            hc = jnp.maximum(
                jnp.dot(x_ref[...], w1_buf[slot],
                        preferred_element_type=jnp.float32),
                0.0,
            )
            pc = jnp.dot(hc, w2_buf[slot], preferred_element_type=jnp.float32)
            if kc == 0:
                out_ref[...] = pc
            else:
                out_ref[...] += pc
            if kc + 1 < nk:
                pending = nxt

        comm_ref[0] = out_ref[...]

        for hop in range(N_DEV - 1):
            s = hop % 2
            r = (hop + 1) % 2
            rdma = pltpu.make_async_remote_copy(
                src_ref=comm_ref.at[s],
                dst_ref=comm_ref.at[r],
                send_sem=send_sems.at[s],
                recv_sem=recv_sems.at[r],
                device_id=(right,),
                device_id_type=pl.DeviceIdType.MESH,
            )
            rdma.start()
            rdma.wait()
            out_ref[...] += comm_ref[r]

    return pl.pallas_call(
        body,
        out_shape=jax.ShapeDtypeStruct((m, n), jnp.float32),
        in_specs=[
            pl.BlockSpec(memory_space=pltpu.VMEM),
            pl.BlockSpec(memory_space=pl.ANY),
            pl.BlockSpec(memory_space=pl.ANY),
        ],
        out_specs=pl.BlockSpec(memory_space=pltpu.VMEM),
        scratch_shapes=[
            pltpu.VMEM((2, k_in, KC), jnp.float32),
            pltpu.VMEM((2, KC, n), jnp.float32),
            pltpu.SemaphoreType.DMA((2, 2)),
            pltpu.VMEM((2, m, n), jnp.float32),
            pltpu.SemaphoreType.DMA((2,)),
            pltpu.SemaphoreType.DMA((2,)),
        ],
        compiler_params=pltpu.CompilerParams(collective_id=0),
    )(x, W1, W2)


# device time: 158869 ns/iter; 2.4074x vs baseline; 2.4074x over previous
import jax
import jax.numpy as jnp
from jax import lax
from jax.experimental import pallas as pl
from jax.experimental.pallas import tpu as pltpu

N_DEV = 4
KC = 256


def kernel(x, W1, W2):
    m, k_in = x.shape
    _, h_per = W1.shape
    _, n = W2.shape
    nk = h_per // KC
    m2 = m // 2
    cm = m2 // N_DEV

    def body(x_ref, w1_hbm, w2_hbm, out_ref,
             w1_buf, w2_buf, load_sems, comm_ref, send_sems, recv_sems):
        my = lax.axis_index("i")
        left = (my + N_DEV - 1) % N_DEV
        right = (my + 1) % N_DEV

        barrier_sem = pltpu.get_barrier_semaphore()
        for nbr in (left, right):
            pl.semaphore_signal(
                barrier_sem, inc=1,
                device_id=(nbr,), device_id_type=pl.DeviceIdType.MESH,
            )
        pl.semaphore_wait(barrier_sem, 2)

        def load(kc, slot):
            c1 = pltpu.make_async_copy(
                w1_hbm.at[:, pl.ds(kc * KC, KC)], w1_buf.at[slot],
                load_sems.at[slot, 0])
            c2 = pltpu.make_async_copy(
                w2_hbm.at[pl.ds(kc * KC, KC), :], w2_buf.at[slot],
                load_sems.at[slot, 1])
            c1.start()
            c2.start()
            return c1, c2

        pending = load(0, 0)
        for kc in range(nk):
            slot = kc % 2
            if kc + 1 < nk:
                nxt = load(kc + 1, (kc + 1) % 2)
            pending[0].wait()
            pending[1].wait()
            hc = jnp.maximum(
                jnp.dot(x_ref[...], w1_buf[slot],
                        preferred_element_type=jnp.float32),
                0.0,
            )
            pc = jnp.dot(hc, w2_buf[slot], preferred_element_type=jnp.float32)
            if kc == 0:
                out_ref[...] = pc
            else:
                out_ref[...] += pc
            if kc + 1 < nk:
                pending = nxt

        dirs = ((1, right, 0), (-1, left, 1))

        def rows(hf, c):
            return pl.ds(hf * m2 + c * cm, cm)

        for s in range(N_DEV - 1):
            ss, rs = s % 2, (s + 1) % 2
            rdmas = []
            for d, tgt, hf in dirs:
                if s == 0:
                    send_c = (my - d * s) % N_DEV
                    comm_ref[hf, ss] = out_ref[rows(hf, send_c), :]
                rdma = pltpu.make_async_remote_copy(
                    src_ref=comm_ref.at[hf, ss],
                    dst_ref=comm_ref.at[hf, rs],
                    send_sem=send_sems.at[hf, ss],
                    recv_sem=recv_sems.at[hf, rs],
                    device_id=(tgt,),
                    device_id_type=pl.DeviceIdType.MESH,
                )
                rdma.start()
                rdmas.append(rdma)
            for rdma in rdmas:
                rdma.wait()
            for d, tgt, hf in dirs:
                recv_c = (my - d * (s + 1)) % N_DEV
                comm_ref[hf, rs] += out_ref[rows(hf, recv_c), :]

        own_slot = (N_DEV - 1) % 2
        for d, tgt, hf in dirs:
            out_ref[rows(hf, (my + d) % N_DEV), :] = comm_ref[hf, own_slot]

        for s in range(N_DEV - 1):
            t = (N_DEV - 1) + s
            ss, rs = t % 2, (t + 1) % 2
            rdmas = []
            for d, tgt, hf in dirs:
                rdma = pltpu.make_async_remote_copy(
                    src_ref=comm_ref.at[hf, ss],
                    dst_ref=comm_ref.at[hf, rs],
                    send_sem=send_sems.at[hf, ss],
                    recv_sem=recv_sems.at[hf, rs],
                    device_id=(tgt,),
                    device_id_type=pl.DeviceIdType.MESH,
                )
                rdma.start()
                rdmas.append(rdma)
            for rdma in rdmas:
                rdma.wait()
            for d, tgt, hf in dirs:
                recv_c = (my - d * s) % N_DEV
                out_ref[rows(hf, recv_c), :] = comm_ref[hf, rs]

    return pl.pallas_call(
        body,
        out_shape=jax.ShapeDtypeStruct((m, n), jnp.float32),
        in_specs=[
            pl.BlockSpec(memory_space=pltpu.VMEM),
            pl.BlockSpec(memory_space=pl.ANY),
            pl.BlockSpec(memory_space=pl.ANY),
        ],
        out_specs=pl.BlockSpec(memory_space=pltpu.VMEM),
        scratch_shapes=[
            pltpu.VMEM((2, k_in, KC), jnp.float32),
            pltpu.VMEM((2, KC, n), jnp.float32),
            pltpu.SemaphoreType.DMA((2, 2)),
            pltpu.VMEM((2, 2, cm, n), jnp.float32),
            pltpu.SemaphoreType.DMA((2, 2)),
            pltpu.SemaphoreType.DMA((2, 2)),
        ],
        compiler_params=pltpu.CompilerParams(collective_id=0),
    )(x, W1, W2)


# device time: 121515 ns/iter; 3.1475x vs baseline; 1.3074x over previous
import jax
import jax.numpy as jnp
from jax import lax
from jax.experimental import pallas as pl
from jax.experimental.pallas import tpu as pltpu

N_DEV = 4
KC = 256


def kernel(x, W1, W2):
    m, k_in = x.shape
    _, h_per = W1.shape
    _, n = W2.shape
    nk = h_per // KC
    m2 = m // 2
    cm = m2 // N_DEV

    def body(x_ref, w1_hbm, w2_hbm, out_ref,
             w1_buf, w2_buf, load_sems, comm_ref, send_sems, recv_sems):
        my = lax.axis_index("i")
        left = (my + N_DEV - 1) % N_DEV
        right = (my + 1) % N_DEV

        barrier_sem = pltpu.get_barrier_semaphore()
        for nbr in (left, right):
            pl.semaphore_signal(
                barrier_sem, inc=1,
                device_id=(nbr,), device_id_type=pl.DeviceIdType.MESH,
            )
        pl.semaphore_wait(barrier_sem, 2)

        def load(kc, slot):
            c1 = pltpu.make_async_copy(
                w1_hbm.at[:, pl.ds(kc * KC, KC)], w1_buf.at[slot],
                load_sems.at[slot, 0])
            c2 = pltpu.make_async_copy(
                w2_hbm.at[pl.ds(kc * KC, KC), :], w2_buf.at[slot],
                load_sems.at[slot, 1])
            c1.start()
            c2.start()
            return c1, c2

        pending = load(0, 0)
        for kc in range(nk):
            slot = kc % 2
            if kc + 1 < nk:
                nxt = load(kc + 1, (kc + 1) % 2)
            pending[0].wait()
            pending[1].wait()
            hc = jnp.maximum(
                jnp.dot(x_ref[...], w1_buf[slot],
                        preferred_element_type=jnp.float32),
                0.0,
            )
            pc = jnp.dot(hc, w2_buf[slot], preferred_element_type=jnp.float32)
            if kc == 0:
                out_ref[...] = pc
            else:
                out_ref[...] += pc
            if kc + 1 < nk:
                pending = nxt

        dirs = ((1, right, 0), (-1, left, 1))

        def rows(hf, c):
            return pl.ds(hf * m2 + c * cm, cm)

        for s in range(N_DEV - 1):
            ss, rs = s % 2, (s + 1) % 2
            rdmas = []
            for d, tgt, hf in dirs:
                if s == 0:
                    send_c = (my - d * s) % N_DEV
                    comm_ref[hf, ss] = out_ref[rows(hf, send_c), :].astype(
                        jnp.bfloat16)
                rdma = pltpu.make_async_remote_copy(
                    src_ref=comm_ref.at[hf, ss],
                    dst_ref=comm_ref.at[hf, rs],
                    send_sem=send_sems.at[hf, ss],
                    recv_sem=recv_sems.at[hf, rs],
                    device_id=(tgt,),
                    device_id_type=pl.DeviceIdType.MESH,
                )
                rdma.start()
                rdmas.append(rdma)
            for rdma in rdmas:
                rdma.wait()
            for d, tgt, hf in dirs:
                recv_c = (my - d * (s + 1)) % N_DEV
                comm_ref[hf, rs] = (
                    comm_ref[hf, rs].astype(jnp.float32)
                    + out_ref[rows(hf, recv_c), :]
                ).astype(jnp.bfloat16)

        own_slot = (N_DEV - 1) % 2
        for d, tgt, hf in dirs:
            out_ref[rows(hf, (my + d) % N_DEV), :] = comm_ref[
                hf, own_slot].astype(jnp.float32)

        for s in range(N_DEV - 1):
            t = (N_DEV - 1) + s
            ss, rs = t % 2, (t + 1) % 2
            rdmas = []
            for d, tgt, hf in dirs:
                rdma = pltpu.make_async_remote_copy(
                    src_ref=comm_ref.at[hf, ss],
                    dst_ref=comm_ref.at[hf, rs],
                    send_sem=send_sems.at[hf, ss],
                    recv_sem=recv_sems.at[hf, rs],
                    device_id=(tgt,),
                    device_id_type=pl.DeviceIdType.MESH,
                )
                rdma.start()
                rdmas.append(rdma)
            for rdma in rdmas:
                rdma.wait()
            for d, tgt, hf in dirs:
                recv_c = (my - d * s) % N_DEV
                out_ref[rows(hf, recv_c), :] = comm_ref[hf, rs].astype(
                    jnp.float32)

    return pl.pallas_call(
        body,
        out_shape=jax.ShapeDtypeStruct((m, n), jnp.float32),
        in_specs=[
            pl.BlockSpec(memory_space=pltpu.VMEM),
            pl.BlockSpec(memory_space=pl.ANY),
            pl.BlockSpec(memory_space=pl.ANY),
        ],
        out_specs=pl.BlockSpec(memory_space=pltpu.VMEM),
        scratch_shapes=[
            pltpu.VMEM((2, k_in, KC), jnp.float32),
            pltpu.VMEM((2, KC, n), jnp.float32),
            pltpu.SemaphoreType.DMA((2, 2)),
            pltpu.VMEM((2, 2, cm, n), jnp.bfloat16),
            pltpu.SemaphoreType.DMA((2, 2)),
            pltpu.SemaphoreType.DMA((2, 2)),
        ],
        compiler_params=pltpu.CompilerParams(collective_id=0),
    )(x, W1, W2)


# device time: 95083 ns/iter; 4.0224x vs baseline; 1.2780x over previous
import jax
import jax.numpy as jnp
from jax import lax
from jax.experimental import pallas as pl
from jax.experimental.pallas import tpu as pltpu

N_DEV = 4
KC = 512


def kernel(x, W1, W2):
    m, k_in = x.shape
    _, h_per = W1.shape
    _, n = W2.shape
    nk = h_per // KC
    m2 = m // 2
    cm = m2 // N_DEV

    def body(x_ref, w1_hbm, w2_hbm, out_ref,
             x_bf, w1_buf, w2_buf, load_sems, comm_ref, send_sems, recv_sems):
        my = lax.axis_index("i")
        left = (my + N_DEV - 1) % N_DEV
        right = (my + 1) % N_DEV

        barrier_sem = pltpu.get_barrier_semaphore()
        for nbr in (left, right):
            pl.semaphore_signal(
                barrier_sem, inc=1,
                device_id=(nbr,), device_id_type=pl.DeviceIdType.MESH,
            )
        pl.semaphore_wait(barrier_sem, 2)

        def load(kc, slot):
            c1 = pltpu.make_async_copy(
                w1_hbm.at[:, pl.ds(kc * KC, KC)], w1_buf.at[slot],
                load_sems.at[slot, 0])
            c2 = pltpu.make_async_copy(
                w2_hbm.at[pl.ds(kc * KC, KC), :], w2_buf.at[slot],
                load_sems.at[slot, 1])
            c1.start()
            c2.start()
            return c1, c2

        pending = load(0, 0)
        x_bf[...] = x_ref[...].astype(jnp.bfloat16)
        for kc in range(nk):
            slot = kc % 2
            if kc + 1 < nk:
                nxt = load(kc + 1, (kc + 1) % 2)
            pending[0].wait()
            pending[1].wait()
            hc = jnp.maximum(
                jnp.dot(x_bf[...], w1_buf[slot].astype(jnp.bfloat16),
                        preferred_element_type=jnp.float32),
                0.0,
            ).astype(jnp.bfloat16)
            pc = jnp.dot(hc, w2_buf[slot].astype(jnp.bfloat16),
                         preferred_element_type=jnp.float32)
            if kc == 0:
                out_ref[...] = pc
            else:
                out_ref[...] += pc
            if kc + 1 < nk:
                pending = nxt

        dirs = ((1, right, 0), (-1, left, 1))

        def rows(hf, c):
            return pl.ds(hf * m2 + c * cm, cm)

        for s in range(N_DEV - 1):
            ss, rs = s % 2, (s + 1) % 2
            rdmas = []
            for d, tgt, hf in dirs:
                if s == 0:
                    send_c = (my - d * s) % N_DEV
                    comm_ref[hf, ss] = out_ref[rows(hf, send_c), :].astype(
                        jnp.bfloat16)
                rdma = pltpu.make_async_remote_copy(
                    src_ref=comm_ref.at[hf, ss],
                    dst_ref=comm_ref.at[hf, rs],
                    send_sem=send_sems.at[hf, ss],
                    recv_sem=recv_sems.at[hf, rs],
                    device_id=(tgt,),
                    device_id_type=pl.DeviceIdType.MESH,
                )
                rdma.start()
                rdmas.append(rdma)
            for rdma in rdmas:
                rdma.wait()
            for d, tgt, hf in dirs:
                recv_c = (my - d * (s + 1)) % N_DEV
                comm_ref[hf, rs] = (
                    comm_ref[hf, rs].astype(jnp.float32)
                    + out_ref[rows(hf, recv_c), :]
                ).astype(jnp.bfloat16)

        own_slot = (N_DEV - 1) % 2
        for d, tgt, hf in dirs:
            out_ref[rows(hf, (my + d) % N_DEV), :] = comm_ref[
                hf, own_slot].astype(jnp.float32)

        for s in range(N_DEV - 1):
            t = (N_DEV - 1) + s
            ss, rs = t % 2, (t + 1) % 2
            rdmas = []
            for d, tgt, hf in dirs:
                rdma = pltpu.make_async_remote_copy(
                    src_ref=comm_ref.at[hf, ss],
                    dst_ref=comm_ref.at[hf, rs],
                    send_sem=send_sems.at[hf, ss],
                    recv_sem=recv_sems.at[hf, rs],
                    device_id=(tgt,),
                    device_id_type=pl.DeviceIdType.MESH,
                )
                rdma.start()
                rdmas.append(rdma)
            for rdma in rdmas:
                rdma.wait()
            for d, tgt, hf in dirs:
                recv_c = (my - d * s) % N_DEV
                out_ref[rows(hf, recv_c), :] = comm_ref[hf, rs].astype(
                    jnp.float32)

    return pl.pallas_call(
        body,
        out_shape=jax.ShapeDtypeStruct((m, n), jnp.float32),
        in_specs=[
            pl.BlockSpec(memory_space=pltpu.VMEM),
            pl.BlockSpec(memory_space=pl.ANY),
            pl.BlockSpec(memory_space=pl.ANY),
        ],
        out_specs=pl.BlockSpec(memory_space=pltpu.VMEM),
        scratch_shapes=[
            pltpu.VMEM((m, k_in), jnp.bfloat16),
            pltpu.VMEM((2, k_in, KC), jnp.float32),
            pltpu.VMEM((2, KC, n), jnp.float32),
            pltpu.SemaphoreType.DMA((2, 2)),
            pltpu.VMEM((2, 2, cm, n), jnp.bfloat16),
            pltpu.SemaphoreType.DMA((2, 2)),
            pltpu.SemaphoreType.DMA((2, 2)),
        ],
        compiler_params=pltpu.CompilerParams(collective_id=0),
    )(x, W1, W2)
